# baseline (speedup 1.0000x reference)
import functools
import jax
import jax.numpy as jnp
from jax import lax
from jax.experimental import pallas as pl
from jax.experimental.pallas import tpu as pltpu
from jax.experimental.pallas import tpu_sc as plsc

_B, _N, _CIN, _COUT = 8, 1000000, 4, 3
_NW = 32
_CL = 1024
_NU = _N // _CL
_ROUNDS = _NU // _NW
_NEXTRA = _NU - _ROUNDS * _NW
_TAIL0 = _NU * _CL
_TAILLEN = _N - _TAIL0
_TAILW = _NW - 1
_PSRC = (3, 1, 2)

_mesh = plsc.VectorSubcoreMesh(core_axis_name="c", subcore_axis_name="s")


@functools.partial(
    pl.kernel,
    mesh=_mesh,
    out_type=jax.ShapeDtypeStruct((_COUT, _B, _N), jnp.float32),
    scratch_types=[
        pltpu.VMEM((_B, _CIN, _CL), jnp.float32),
        pltpu.VMEM((_B, _CIN, _CL), jnp.float32),
        pltpu.VMEM((_B, _CIN, _TAILLEN), jnp.float32),
        pltpu.SemaphoreType.DMA,
        pltpu.SemaphoreType.DMA,
    ],
    compiler_params=pltpu.CompilerParams(needs_layout_passes=False),
)
def _sph2vec_sc(xt_hbm, out_hbm, buf0, buf1, tail_v, sem0, sem1):
    cid = lax.axis_index("c")
    sid = lax.axis_index("s")
    wid = sid * 2 + cid
    bufs = (buf0, buf1)
    sems = (sem0, sem1)

    def chunk_of(u):
        return u * _NW + wid

    def fire_outs(buf, sem, n0):
        for ch in range(_COUT):
            pltpu.async_copy(
                buf.at[:, _PSRC[ch], :], out_hbm.at[ch, :, pl.ds(n0, _CL)], sem
            )

    def drain_outs(buf, sem, n0):
        for ch in range(_COUT):
            pltpu.make_async_copy(
                buf.at[:, _PSRC[ch], :], out_hbm.at[ch, :, pl.ds(n0, _CL)], sem
            ).wait()

    for u in range(_ROUNDS):
        b = u & 1
        if u >= 2:
            drain_outs(bufs[b], sems[b], chunk_of(u - 2) * _CL)
        pltpu.sync_copy(xt_hbm.at[:, :, pl.ds(chunk_of(u) * _CL, _CL)], bufs[b])
        fire_outs(bufs[b], sems[b], chunk_of(u) * _CL)

    extra_n0 = (_ROUNDS * _NW + wid) * _CL

    @pl.when(wid < _NEXTRA)
    def _extra():
        drain_outs(bufs[0], sems[0], chunk_of(_ROUNDS - 2) * _CL)
        pltpu.sync_copy(xt_hbm.at[:, :, pl.ds(extra_n0, _CL)], bufs[0])
        fire_outs(bufs[0], sems[0], extra_n0)
        drain_outs(bufs[0], sems[0], extra_n0)

    @pl.when(wid >= _NEXTRA)
    def _no_extra():
        drain_outs(bufs[0], sems[0], chunk_of(_ROUNDS - 2) * _CL)

    drain_outs(bufs[1], sems[1], chunk_of(_ROUNDS - 1) * _CL)

    @pl.when(wid == _TAILW)
    def _tail():
        pltpu.sync_copy(xt_hbm.at[:, :, pl.ds(_TAIL0, _TAILLEN)], tail_v)
        for ch in range(_COUT):
            pltpu.sync_copy(
                tail_v.at[:, _PSRC[ch], :],
                out_hbm.at[ch, :, pl.ds(_TAIL0, _TAILLEN)],
            )


def kernel(x):
    xt = jnp.transpose(x, (0, 2, 1))
    out_t = _sph2vec_sc(xt)
    return jnp.transpose(out_t, (1, 2, 0))


# speedup vs baseline: 1.1594x; 1.1594x over previous
import functools
import jax
import jax.numpy as jnp
from jax import lax
from jax.experimental import pallas as pl
from jax.experimental.pallas import tpu as pltpu
from jax.experimental.pallas import tpu_sc as plsc

_B, _N, _CIN, _COUT = 8, 1000000, 4, 3
_NW = 32
_CL = 2048
_NU = _N // _CL
_TAIL0 = _NU * _CL
_TAILLEN = _N - _TAIL0
_TAILW = _NW - 1
_PSRC = (3, 1, 2)

_mesh = plsc.VectorSubcoreMesh(core_axis_name="c", subcore_axis_name="s")


@functools.partial(
    pl.kernel,
    mesh=_mesh,
    out_type=jax.ShapeDtypeStruct((_COUT, _B, _N), jnp.float32),
    scratch_types=[
        pltpu.VMEM((_B, _CIN, _CL), jnp.float32),
        pltpu.VMEM((_B, _CIN, _TAILLEN), jnp.float32),
    ],
    compiler_params=pltpu.CompilerParams(needs_layout_passes=False),
)
def _sph2vec_sc(xt_hbm, out_hbm, in_v, tail_v):
    cid = lax.axis_index("c")
    sid = lax.axis_index("s")
    wid = sid * 2 + cid
    n_units = (_NU - wid + _NW - 1) // _NW

    def unit_body(u, carry):
        n0 = (u * _NW + wid) * _CL
        pltpu.sync_copy(xt_hbm.at[:, :, pl.ds(n0, _CL)], in_v)
        for ch in range(_COUT):
            pltpu.sync_copy(
                in_v.at[:, _PSRC[ch], :], out_hbm.at[ch, :, pl.ds(n0, _CL)]
            )
        return carry

    lax.fori_loop(0, n_units, unit_body, 0)

    @pl.when(wid == _TAILW)
    def _tail():
        pltpu.sync_copy(xt_hbm.at[:, :, pl.ds(_TAIL0, _TAILLEN)], tail_v)
        for ch in range(_COUT):
            pltpu.sync_copy(
                tail_v.at[:, _PSRC[ch], :],
                out_hbm.at[ch, :, pl.ds(_TAIL0, _TAILLEN)],
            )


def kernel(x):
    xt = jnp.transpose(x, (0, 2, 1))
    out_t = _sph2vec_sc(xt)
    return jnp.transpose(out_t, (1, 2, 0))


# speedup vs baseline: 1.1943x; 1.0301x over previous
import functools
import jax
import jax.numpy as jnp
from jax import lax
from jax.experimental import pallas as pl
from jax.experimental.pallas import tpu as pltpu
from jax.experimental.pallas import tpu_sc as plsc

_B, _N, _CIN, _COUT = 8, 1000000, 4, 3
_NW = 32
_CL = 3456
_NU = _N // _CL
_TAIL0 = _NU * _CL
_TAILA = (_N - _TAIL0) // 128 * 128
_TAILB0 = _TAIL0 + _TAILA
_TAILBLEN = _N - _TAILB0
_TAILW = _NW - 1
_PSRC = (3, 1, 2)

_mesh = plsc.VectorSubcoreMesh(core_axis_name="c", subcore_axis_name="s")


@functools.partial(
    pl.kernel,
    mesh=_mesh,
    out_type=jax.ShapeDtypeStruct((_COUT, _B, _N), jnp.float32),
    scratch_types=[
        pltpu.VMEM((_B, _CIN, _CL), jnp.float32),
        pltpu.VMEM((_B, _CIN, _TAILBLEN), jnp.float32),
    ],
    compiler_params=pltpu.CompilerParams(needs_layout_passes=False),
)
def _sph2vec_sc(xt_hbm, out_hbm, in_v, tail_v):
    cid = lax.axis_index("c")
    sid = lax.axis_index("s")
    wid = sid * 2 + cid
    n_units = (_NU - wid + _NW - 1) // _NW

    def unit_body(u, carry):
        n0 = (u * _NW + wid) * _CL
        pltpu.sync_copy(xt_hbm.at[:, :, pl.ds(n0, _CL)], in_v)
        for ch in range(_COUT):
            pltpu.sync_copy(
                in_v.at[:, _PSRC[ch], :], out_hbm.at[ch, :, pl.ds(n0, _CL)]
            )
        return carry

    lax.fori_loop(0, n_units, unit_body, 0)

    @pl.when(wid == _TAILW)
    def _tail():
        pltpu.sync_copy(
            xt_hbm.at[:, :, pl.ds(_TAIL0, _TAILA)],
            in_v.at[:, :, pl.ds(0, _TAILA)],
        )
        for ch in range(_COUT):
            pltpu.sync_copy(
                in_v.at[:, _PSRC[ch], pl.ds(0, _TAILA)],
                out_hbm.at[ch, :, pl.ds(_TAIL0, _TAILA)],
            )
        pltpu.sync_copy(xt_hbm.at[:, :, pl.ds(_TAILB0, _TAILBLEN)], tail_v)
        for ch in range(_COUT):
            pltpu.sync_copy(
                tail_v.at[:, _PSRC[ch], :],
                out_hbm.at[ch, :, pl.ds(_TAILB0, _TAILBLEN)],
            )


def kernel(x):
    xt = jnp.transpose(x, (0, 2, 1))
    out_t = _sph2vec_sc(xt)
    return jnp.transpose(out_t, (1, 2, 0))
